# Initial kernel scaffold; baseline (speedup 1.0000x reference)
import jax
import jax.numpy as jnp
from jax.experimental import pallas as pl


def kernel(x):
    raise NotImplementedError("write your pallas kernel here")


# speedup vs baseline: 25.0774x; 25.0774x over previous
import jax
import jax.numpy as jnp
from jax.experimental import pallas as pl
from jax.experimental.pallas import tpu as pltpu

_N = 524288
_NLOW = 262144
_R = 4096
_C = 128
_TOP = -(2 ** 31)


def _erf(x):
    sgn = jnp.where(x < 0.0, -1.0, 1.0)
    a = jnp.abs(x)
    t = 1.0 / (1.0 + 0.3275911 * a)
    poly = t * (0.254829592 + t * (-0.284496736 + t * (1.421413741
           + t * (-1.453152027 + t * 1.061405429))))
    y = 1.0 - poly * jnp.exp(-a * a)
    return sgn * y


def _binrisk(mu0, mu1, v0, v1):
    sq2 = jnp.sqrt(jnp.float32(2.0))
    sigma0 = jnp.sqrt(v0)
    sigma1 = jnp.sqrt(v1)
    inv_sqrt2pi = jnp.float32(1.0) / jnp.sqrt(jnp.float32(2.0) * jnp.pi)
    mor0 = jnp.exp(-0.5 * ((-1.0 - mu0) / sigma0) ** 2) * inv_sqrt2pi / sigma0
    mor1 = jnp.exp(-0.5 * ((1.0 - mu1) / sigma1) ** 2) * inv_sqrt2pi / sigma1
    m = mu0 + 1.0
    r = 0.25 * m
    mm = (-mu0 - 1.0) / (sq2 * sigma0)
    r = r * (1.0 - _erf(mm))
    r = r + 0.5 * v0 * mor0
    m3 = 1.0 - mu1
    term3 = 0.25 * m3 * (1.0 + _erf(m3 / (sq2 * sigma1)))
    r = r + term3
    r = r + 0.5 * v1 * mor1
    return r


def _body(x_ref, out_ref):
    xv = x_ref[...]
    k = jax.lax.bitcast_convert_type(xv, jnp.int32)
    m = k >> 31
    skey = k ^ (m & jnp.int32(0x7FFFFFFF))

    total_s = jnp.sum(xv)
    total_ss = jnp.sum(xv * xv)

    def step(i, carry):
        p, clt = carry
        b = 31 - i
        cand = p | (jnp.int32(1) << b)
        scand = cand ^ jnp.int32(_TOP)
        cnt = jnp.sum((skey < scand).astype(jnp.int32))
        take = cnt <= _NLOW
        return (jnp.where(take, cand, p), jnp.where(take, cnt, clt))

    p, clt = jax.lax.fori_loop(0, 32, step, (jnp.int32(0), jnp.int32(0)))

    sk = p ^ jnp.int32(_TOP)
    mlt = skey < sk
    meq = skey == sk
    sum_lt = jnp.sum(jnp.where(mlt, xv, 0.0))
    ss_lt = jnp.sum(jnp.where(mlt, xv * xv, 0.0))
    t = jnp.max(jnp.where(meq, xv, -jnp.inf))

    nlow = jnp.float32(_NLOW)
    nhigh = jnp.float32(_N - _NLOW)
    fill = nlow - clt.astype(jnp.float32)
    sum_low = sum_lt + fill * t
    ss_low = ss_lt + fill * t * t
    sum_high = total_s - sum_low
    ss_high = total_ss - ss_low

    mu0 = sum_low / nlow
    mu1 = sum_high / nhigh
    var0 = (ss_low - sum_low * mu0) / (nlow - 1.0)
    var1 = (ss_high - sum_high * mu1) / (nhigh - 1.0)
    sig0 = jnp.sqrt(var0)
    sig1 = jnp.sqrt(var1)

    out_ref[0] = _binrisk(mu0, mu1, sig0, sig1) + t * t


def kernel(x):
    x2 = x.reshape(_R, _C)
    res = pl.pallas_call(
        _body,
        in_specs=[pl.BlockSpec(memory_space=pltpu.VMEM)],
        out_specs=pl.BlockSpec(memory_space=pltpu.SMEM),
        out_shape=jax.ShapeDtypeStruct((1,), jnp.float32),
    )(x2)
    return res[0]
